# initial kernel scaffold (unmeasured)
import jax
import jax.numpy as jnp
from jax import lax
from jax.experimental import pallas as pl
from jax.experimental.pallas import tpu as pltpu


def kernel(
    x,
):
    def body(*refs):
        pass

    out_shape = jax.ShapeDtypeStruct(..., jnp.float32)
    return pl.pallas_call(body, out_shape=out_shape)(...)


# baseline (device time: 21218 ns/iter reference)
import jax
import jax.numpy as jnp
from jax import lax
from jax.experimental import pallas as pl
from jax.experimental.pallas import tpu as pltpu


def kernel(x):
    m_per, n = x.shape
    m_glob = 2 * m_per
    n_per = n // 2
    half = m_per // 2

    def body(x_ref, out_ref, send_buf, rx_buf, ry_buf, sems):
        mx = lax.axis_index("x")
        my = lax.axis_index("y")
        px = 1 - mx
        py = 1 - my

        barrier_sem = pltpu.get_barrier_semaphore()
        pl.semaphore_signal(
            barrier_sem, inc=1,
            device_id=(px, my), device_id_type=pl.DeviceIdType.MESH,
        )
        pl.semaphore_signal(
            barrier_sem, inc=1,
            device_id=(mx, py), device_id_type=pl.DeviceIdType.MESH,
        )
        pl.semaphore_wait(barrier_sem, 2)

        send_buf[...] = x_ref[
            pl.ds(my * half, half), pl.ds(px * n_per, n_per)
        ].astype(jnp.bfloat16)

        rdma_x = pltpu.make_async_remote_copy(
            src_ref=send_buf,
            dst_ref=rx_buf,
            send_sem=sems.at[0],
            recv_sem=sems.at[1],
            device_id=(px, my),
            device_id_type=pl.DeviceIdType.MESH,
        )
        rdma_x.start()

        out_ref[pl.ds(mx * m_per, m_per), :] = x_ref[:, pl.ds(mx * n_per, n_per)]

        rdma_x.wait()

        rdma_y = pltpu.make_async_remote_copy(
            src_ref=rx_buf,
            dst_ref=ry_buf,
            send_sem=sems.at[2],
            recv_sem=sems.at[3],
            device_id=(mx, py),
            device_id_type=pl.DeviceIdType.MESH,
        )
        rdma_y.start()

        out_ref[pl.ds(px * m_per + my * half, half), :] = rx_buf[...].astype(
            jnp.float32
        )

        rdma_y.wait()
        out_ref[pl.ds(px * m_per + py * half, half), :] = ry_buf[...].astype(
            jnp.float32
        )

    return pl.pallas_call(
        body,
        out_shape=jax.ShapeDtypeStruct((m_glob, n_per), x.dtype),
        in_specs=[pl.BlockSpec(memory_space=pltpu.VMEM)],
        out_specs=pl.BlockSpec(memory_space=pltpu.VMEM),
        scratch_shapes=[
            pltpu.VMEM((half, n_per), jnp.bfloat16),
            pltpu.VMEM((half, n_per), jnp.bfloat16),
            pltpu.VMEM((half, n_per), jnp.bfloat16),
            pltpu.SemaphoreType.DMA((4,)),
        ],
        compiler_params=pltpu.CompilerParams(collective_id=0),
    )(x)


# device time: 16776 ns/iter; 1.2648x vs baseline; 1.2648x over previous
import jax
import jax.numpy as jnp
from jax import lax
from jax.experimental import pallas as pl
from jax.experimental.pallas import tpu as pltpu

C = 8


def kernel(x):
    m_per, n = x.shape
    m_glob = 2 * m_per
    n_per = n // 2
    half = m_per // 2
    rows = half // C

    def body(x_ref, out_ref, send_buf, rx_buf, ry_buf, sx, rx_sem, sy, ry_sem):
        mx = lax.axis_index("x")
        my = lax.axis_index("y")
        px = 1 - mx
        py = 1 - my

        barrier_sem = pltpu.get_barrier_semaphore()
        pl.semaphore_signal(
            barrier_sem, inc=1,
            device_id=(px, my), device_id_type=pl.DeviceIdType.MESH,
        )
        pl.semaphore_signal(
            barrier_sem, inc=1,
            device_id=(mx, py), device_id_type=pl.DeviceIdType.MESH,
        )
        pl.semaphore_wait(barrier_sem, 2)

        def x_rdma(i):
            return pltpu.make_async_remote_copy(
                src_ref=send_buf.at[i],
                dst_ref=rx_buf.at[i],
                send_sem=sx.at[i],
                recv_sem=rx_sem.at[i],
                device_id=(px, my),
                device_id_type=pl.DeviceIdType.MESH,
            )

        def y_rdma(i):
            return pltpu.make_async_remote_copy(
                src_ref=rx_buf.at[i],
                dst_ref=ry_buf.at[i],
                send_sem=sy.at[i],
                recv_sem=ry_sem.at[i],
                device_id=(mx, py),
                device_id_type=pl.DeviceIdType.MESH,
            )

        for i in range(C):
            send_buf[i, :, :] = x_ref[
                pl.ds(my * half + i * rows, rows), pl.ds(px * n_per, n_per)
            ].astype(jnp.bfloat16)
            x_rdma(i).start()

        out_ref[pl.ds(mx * m_per, m_per), :] = x_ref[:, pl.ds(mx * n_per, n_per)]

        for i in range(C):
            x_rdma(i).wait_recv()
            y_rdma(i).start()
            out_ref[pl.ds(px * m_per + my * half + i * rows, rows), :] = rx_buf[
                i
            ].astype(jnp.float32)

        for i in range(C):
            y_rdma(i).wait_recv()
            out_ref[pl.ds(px * m_per + py * half + i * rows, rows), :] = ry_buf[
                i
            ].astype(jnp.float32)

        for i in range(C):
            x_rdma(i).wait_send()
            y_rdma(i).wait_send()

    return pl.pallas_call(
        body,
        out_shape=jax.ShapeDtypeStruct((m_glob, n_per), x.dtype),
        in_specs=[pl.BlockSpec(memory_space=pltpu.VMEM)],
        out_specs=pl.BlockSpec(memory_space=pltpu.VMEM),
        scratch_shapes=[
            pltpu.VMEM((C, rows, n_per), jnp.bfloat16),
            pltpu.VMEM((C, rows, n_per), jnp.bfloat16),
            pltpu.VMEM((C, rows, n_per), jnp.bfloat16),
            pltpu.SemaphoreType.DMA((C,)),
            pltpu.SemaphoreType.DMA((C,)),
            pltpu.SemaphoreType.DMA((C,)),
            pltpu.SemaphoreType.DMA((C,)),
        ],
        compiler_params=pltpu.CompilerParams(collective_id=0),
    )(x)
